# baseline (speedup 1.0000x reference)
import jax
import jax.numpy as jnp
from jax import lax
from jax.experimental import pallas as pl
from jax.experimental.pallas import tpu as pltpu
from jax.experimental.pallas import tpu_sc as plsc

_NC, _NS, _L = 2, 16, 16
_NW = _NC * _NS
_R, _C = 512, 32768
_RPW = _R // _NW
_CHUNKS = _C // _L
_BEAM = 4
_EOS = 0
_NEG = float("-inf")
_LN2 = 0.6931471805599453


def _gather16(arr, idx):
    return arr.at[idx].get(mode="promise_in_bounds")


def _allmax(x):
    i = lax.iota(jnp.int32, 16)
    for d in (1, 2, 4, 8):
        x = jnp.maximum(x, _gather16(x, i ^ d))
    return x


def _allmin(x):
    i = lax.iota(jnp.int32, 16)
    for d in (1, 2, 4, 8):
        x = jnp.minimum(x, _gather16(x, i ^ d))
    return x


def _allsum(x):
    i = lax.iota(jnp.int32, 16)
    for d in (1, 2, 4, 8):
        x = x + _gather16(x, i ^ d)
    return x


def _ln_vec(s):
    bits = lax.bitcast_convert_type(s, jnp.int32)
    e = ((bits >> 23) & 0xFF) - 127
    mant = lax.bitcast_convert_type((bits & 0x7FFFFF) | 0x3F800000,
                                    jnp.float32)
    u = mant - 1.0
    y = e.astype(jnp.float32) * _LN2 + u * (1.0 + u * (-0.5 + u * (1.0 / 3.0)))
    for _ in range(3):
        y = y + s * jnp.exp(-y) - 1.0
    return y


def _process_row(buf, rlocal, acc):
    lse_a, rtv, rti = acc
    iota16 = lax.iota(jnp.int32, 16)

    def pa(j, m):
        return jnp.maximum(m, buf[pl.ds(j * _L, _L)])
    m = lax.fori_loop(0, _CHUNKS, pa, jnp.full((_L,), _NEG, jnp.float32))
    Msp = _allmax(m)

    def pb(j, c):
        s, t1, t2, t3, t4, i1, i2, i3, i4 = c
        x = buf[pl.ds(j * _L, _L)]
        s = s + jnp.exp(x - Msp)
        idx = j * _L + iota16
        b1 = x > t1
        b2 = x > t2
        b3 = x > t3
        b4 = x > t4
        t4n = jnp.where(b3, t3, jnp.where(b4, x, t4))
        i4n = jnp.where(b3, i3, jnp.where(b4, idx, i4))
        t3n = jnp.where(b2, t2, jnp.where(b3, x, t3))
        i3n = jnp.where(b2, i2, jnp.where(b3, idx, i3))
        t2n = jnp.where(b1, t1, jnp.where(b2, x, t2))
        i2n = jnp.where(b1, i1, jnp.where(b2, idx, i2))
        t1n = jnp.where(b1, x, t1)
        i1n = jnp.where(b1, idx, i1)
        return (s, t1n, t2n, t3n, t4n, i1n, i2n, i3n, i4n)

    zf = jnp.zeros((_L,), jnp.float32)
    nf = jnp.full((_L,), _NEG, jnp.float32)
    zi = jnp.zeros((_L,), jnp.int32)
    s, t1, t2, t3, t4, i1, i2, i3, i4 = lax.fori_loop(
        0, _CHUNKS, pb, (zf, nf, nf, nf, nf, zi, zi, zi, zi))

    S = _allsum(s)
    lse = _ln_vec(S) + Msp
    here = iota16 == rlocal
    lse_a = jnp.where(here, lse, lse_a)

    rtv_n = list(rtv)
    rti_n = list(rti)
    for k in range(4):
        mk = _allmax(t1)
        isel = _allmin(jnp.where(t1 == mk, i1, _C))
        rtv_n[k] = jnp.where(here, mk, rtv_n[k])
        rti_n[k] = jnp.where(here, isel, rti_n[k])
        w = (t1 == mk) & (i1 == isel)
        t1 = jnp.where(w, t2, t1)
        i1 = jnp.where(w, i2, i1)
        t2 = jnp.where(w, t3, t2)
        i2 = jnp.where(w, i3, i2)
        t3 = jnp.where(w, t4, t3)
        i3 = jnp.where(w, i4, i3)
        t4 = jnp.where(w, nf, t4)
    return (lse_a, tuple(rtv_n), tuple(rti_n))


def _sc_body(x_hbm, pred_hbm, lp_hbm, outv_hbm, outc_hbm, outb_hbm,
             buf0, buf1, lp_v, pred_v, ov, oc, ob, sem0, sem1):
    wid = lax.axis_index("s") * _NC + lax.axis_index("c")
    row0 = wid * _RPW
    pltpu.sync_copy(lp_hbm.at[pl.ds(row0, _RPW)], lp_v)
    pltpu.sync_copy(pred_hbm.at[pl.ds(row0, _RPW)], pred_v)

    iota16 = lax.iota(jnp.int32, 16)
    bufs = (buf0, buf1)
    sems = (sem0, sem1)
    pltpu.async_copy(x_hbm.at[row0], buf0, sem0)

    zf = jnp.zeros((_L,), jnp.float32)
    zi = jnp.zeros((_L,), jnp.int32)

    def group(g, acc):
        for b in range(2):
            r = g * 2 + b
            pltpu.make_async_copy(x_hbm.at[row0 + r], bufs[b], sems[b]).wait()

            @pl.when(r + 1 < _RPW)
            def _():
                pltpu.async_copy(x_hbm.at[row0 + r + 1], bufs[1 - b],
                                 sems[1 - b])
            lse_a, rtv, rti = _process_row(
                bufs[b], r, (acc[0], acc[1:5], acc[5:9]))
            acc = (lse_a,) + rtv + rti
        return acc

    acc = lax.fori_loop(0, _RPW // 2, group,
                        (zf, zf, zf, zf, zf, zi, zi, zi, zi))
    lse_a = acc[0]
    rtv = acc[1:5]
    rti = acc[5:9]
    lp = lp_v[...]
    pred = pred_v[...]

    outv = jnp.zeros((_L,), jnp.float32)
    outc = jnp.zeros((_L,), jnp.int32)
    outb = jnp.zeros((_L,), jnp.int32)
    k_in = iota16 & 3
    for b in range(4):
        rsel = b * 4 + (iota16 >> 2)
        gv = [_gather16(rtv[k], rsel) for k in range(4)]
        gi = [_gather16(rti[k], rsel) for k in range(4)]
        v = jnp.where(k_in == 0, gv[0],
                      jnp.where(k_in == 1, gv[1],
                                jnp.where(k_in == 2, gv[2], gv[3])))
        ci = jnp.where(k_in == 0, gi[0],
                       jnp.where(k_in == 1, gi[1],
                                 jnp.where(k_in == 2, gi[2], gi[3])))
        lpg = _gather16(lp, rsel)
        lseg = _gather16(lse_a, rsel)
        predg = _gather16(pred, rsel)
        fin = predg == _EOS
        candv = jnp.where(fin,
                          jnp.where(k_in == 0, lpg,
                                    jnp.full((16,), _NEG, jnp.float32)),
                          lpg + v - lseg)
        candc = jnp.where(fin, 0, ci)
        flat = ((iota16 >> 2) << 15) | candc
        for k in range(4):
            mk = _allmax(candv)
            fsel = _allmin(jnp.where(candv == mk, flat, _BEAM * _C))
            candv = jnp.where(flat == fsel, _NEG, candv)
            here = iota16 == (b * 4 + k)
            outv = jnp.where(here, mk, outv)
            outc = jnp.where(here, fsel & (_C - 1), outc)
            outb = jnp.where(here, fsel >> 15, outb)

    ov[...] = outv
    oc[...] = outc
    ob[...] = outb
    pltpu.sync_copy(ov, outv_hbm.at[pl.ds(row0, _RPW)])
    pltpu.sync_copy(oc, outc_hbm.at[pl.ds(row0, _RPW)])
    pltpu.sync_copy(ob, outb_hbm.at[pl.ds(row0, _RPW)])


def _make_sc_call(interpret=False):
    mesh = plsc.VectorSubcoreMesh(core_axis_name="c", subcore_axis_name="s",
                                  num_cores=_NC, num_subcores=_NS)
    return pl.kernel(
        _sc_body,
        out_type=[
            jax.ShapeDtypeStruct((_R,), jnp.float32),
            jax.ShapeDtypeStruct((_R,), jnp.int32),
            jax.ShapeDtypeStruct((_R,), jnp.int32),
        ],
        mesh=mesh,
        scratch_types=[
            pltpu.VMEM((_C,), jnp.float32),
            pltpu.VMEM((_C,), jnp.float32),
            pltpu.VMEM((_RPW,), jnp.float32),
            pltpu.VMEM((_RPW,), jnp.int32),
            pltpu.VMEM((_L,), jnp.float32),
            pltpu.VMEM((_L,), jnp.int32),
            pltpu.VMEM((_L,), jnp.int32),
            pltpu.SemaphoreType.DMA,
            pltpu.SemaphoreType.DMA,
        ],
        interpret=interpret,
    )


@jax.jit
def kernel(class_log_probabilities, last_predictions, last_log_probabilities):
    B, beam = last_log_probabilities.shape
    lp_flat = last_log_probabilities.reshape(-1)
    outv, outc, outb = _make_sc_call()(
        class_log_probabilities, last_predictions, lp_flat)
    return (outv.reshape(B, beam), outc.reshape(B, beam),
            outb.reshape(B, beam))


# speedup vs baseline: 1.6188x; 1.6188x over previous
import jax
import jax.numpy as jnp
from jax import lax
from jax.experimental import pallas as pl
from jax.experimental.pallas import tpu as pltpu
from jax.experimental.pallas import tpu_sc as plsc

_NC, _NS, _L = 2, 16, 16
_NW = _NC * _NS
_R, _C = 512, 32768
_RPW = _R // _NW
_CHUNKS = _C // _L
_BEAM = 4
_EOS = 0
_NEG = float("-inf")
_LN2 = 0.6931471805599453


def _gather16(arr, idx):
    return arr.at[idx].get(mode="promise_in_bounds")


def _allmax(x):
    i = lax.iota(jnp.int32, 16)
    for d in (1, 2, 4, 8):
        x = jnp.maximum(x, _gather16(x, i ^ d))
    return x


def _allmin(x):
    i = lax.iota(jnp.int32, 16)
    for d in (1, 2, 4, 8):
        x = jnp.minimum(x, _gather16(x, i ^ d))
    return x


def _allsum(x):
    i = lax.iota(jnp.int32, 16)
    for d in (1, 2, 4, 8):
        x = x + _gather16(x, i ^ d)
    return x


def _ln_vec(s):
    bits = lax.bitcast_convert_type(s, jnp.int32)
    e = ((bits >> 23) & 0xFF) - 127
    mant = lax.bitcast_convert_type((bits & 0x7FFFFF) | 0x3F800000,
                                    jnp.float32)
    u = mant - 1.0
    y = e.astype(jnp.float32) * _LN2 + u * (1.0 + u * (-0.5 + u * (1.0 / 3.0)))
    for _ in range(3):
        y = y + s * jnp.exp(-y) - 1.0
    return y


def _process_row(buf, rlocal, acc):
    lse_a, rtv, rti = acc
    iota16 = lax.iota(jnp.int32, 16)

    def pa(j, m):
        return jnp.maximum(m, buf[pl.ds(j * _L, _L)])
    m = lax.fori_loop(0, _CHUNKS, pa, jnp.full((_L,), _NEG, jnp.float32),
                      unroll=8)
    Msp = _allmax(m)

    def pb(j, c):
        s, t1, t2, t3, t4, i1, i2, i3, i4 = c
        x = buf[pl.ds(j * _L, _L)]
        s = s + jnp.exp(x - Msp)
        idx = j * _L + iota16
        b1 = x > t1
        b2 = x > t2
        b3 = x > t3
        b4 = x > t4
        t4n = jnp.where(b3, t3, jnp.where(b4, x, t4))
        i4n = jnp.where(b3, i3, jnp.where(b4, idx, i4))
        t3n = jnp.where(b2, t2, jnp.where(b3, x, t3))
        i3n = jnp.where(b2, i2, jnp.where(b3, idx, i3))
        t2n = jnp.where(b1, t1, jnp.where(b2, x, t2))
        i2n = jnp.where(b1, i1, jnp.where(b2, idx, i2))
        t1n = jnp.where(b1, x, t1)
        i1n = jnp.where(b1, idx, i1)
        return (s, t1n, t2n, t3n, t4n, i1n, i2n, i3n, i4n)

    zf = jnp.zeros((_L,), jnp.float32)
    nf = jnp.full((_L,), _NEG, jnp.float32)
    zi = jnp.zeros((_L,), jnp.int32)
    s, t1, t2, t3, t4, i1, i2, i3, i4 = lax.fori_loop(
        0, _CHUNKS, pb, (zf, nf, nf, nf, nf, zi, zi, zi, zi), unroll=4)

    S = _allsum(s)
    lse = _ln_vec(S) + Msp
    here = iota16 == rlocal
    lse_a = jnp.where(here, lse, lse_a)

    rtv_n = list(rtv)
    rti_n = list(rti)
    for k in range(4):
        mk = _allmax(t1)
        isel = _allmin(jnp.where(t1 == mk, i1, _C))
        rtv_n[k] = jnp.where(here, mk, rtv_n[k])
        rti_n[k] = jnp.where(here, isel, rti_n[k])
        w = (t1 == mk) & (i1 == isel)
        t1 = jnp.where(w, t2, t1)
        i1 = jnp.where(w, i2, i1)
        t2 = jnp.where(w, t3, t2)
        i2 = jnp.where(w, i3, i2)
        t3 = jnp.where(w, t4, t3)
        i3 = jnp.where(w, i4, i3)
        t4 = jnp.where(w, nf, t4)
    return (lse_a, tuple(rtv_n), tuple(rti_n))


def _sc_body(x_hbm, pred_hbm, lp_hbm, outv_hbm, outc_hbm, outb_hbm,
             buf0, buf1, lp_v, pred_v, ov, oc, ob, sem0, sem1):
    wid = lax.axis_index("s") * _NC + lax.axis_index("c")
    row0 = wid * _RPW
    pltpu.sync_copy(lp_hbm.at[pl.ds(row0, _RPW)], lp_v)
    pltpu.sync_copy(pred_hbm.at[pl.ds(row0, _RPW)], pred_v)

    iota16 = lax.iota(jnp.int32, 16)
    bufs = (buf0, buf1)
    sems = (sem0, sem1)
    pltpu.async_copy(x_hbm.at[row0], buf0, sem0)

    zf = jnp.zeros((_L,), jnp.float32)
    zi = jnp.zeros((_L,), jnp.int32)

    def group(g, acc):
        for b in range(2):
            r = g * 2 + b
            pltpu.make_async_copy(x_hbm.at[row0 + r], bufs[b], sems[b]).wait()

            @pl.when(r + 1 < _RPW)
            def _():
                pltpu.async_copy(x_hbm.at[row0 + r + 1], bufs[1 - b],
                                 sems[1 - b])
            lse_a, rtv, rti = _process_row(
                bufs[b], r, (acc[0], acc[1:5], acc[5:9]))
            acc = (lse_a,) + rtv + rti
        return acc

    acc = lax.fori_loop(0, _RPW // 2, group,
                        (zf, zf, zf, zf, zf, zi, zi, zi, zi))
    lse_a = acc[0]
    rtv = acc[1:5]
    rti = acc[5:9]
    lp = lp_v[...]
    pred = pred_v[...]

    outv = jnp.zeros((_L,), jnp.float32)
    outc = jnp.zeros((_L,), jnp.int32)
    outb = jnp.zeros((_L,), jnp.int32)
    k_in = iota16 & 3
    for b in range(4):
        rsel = b * 4 + (iota16 >> 2)
        gv = [_gather16(rtv[k], rsel) for k in range(4)]
        gi = [_gather16(rti[k], rsel) for k in range(4)]
        v = jnp.where(k_in == 0, gv[0],
                      jnp.where(k_in == 1, gv[1],
                                jnp.where(k_in == 2, gv[2], gv[3])))
        ci = jnp.where(k_in == 0, gi[0],
                       jnp.where(k_in == 1, gi[1],
                                 jnp.where(k_in == 2, gi[2], gi[3])))
        lpg = _gather16(lp, rsel)
        lseg = _gather16(lse_a, rsel)
        predg = _gather16(pred, rsel)
        fin = predg == _EOS
        candv = jnp.where(fin,
                          jnp.where(k_in == 0, lpg,
                                    jnp.full((16,), _NEG, jnp.float32)),
                          lpg + v - lseg)
        candc = jnp.where(fin, 0, ci)
        flat = ((iota16 >> 2) << 15) | candc
        for k in range(4):
            mk = _allmax(candv)
            fsel = _allmin(jnp.where(candv == mk, flat, _BEAM * _C))
            candv = jnp.where(flat == fsel, _NEG, candv)
            here = iota16 == (b * 4 + k)
            outv = jnp.where(here, mk, outv)
            outc = jnp.where(here, fsel & (_C - 1), outc)
            outb = jnp.where(here, fsel >> 15, outb)

    ov[...] = outv
    oc[...] = outc
    ob[...] = outb
    pltpu.sync_copy(ov, outv_hbm.at[pl.ds(row0, _RPW)])
    pltpu.sync_copy(oc, outc_hbm.at[pl.ds(row0, _RPW)])
    pltpu.sync_copy(ob, outb_hbm.at[pl.ds(row0, _RPW)])


def _make_sc_call(interpret=False):
    mesh = plsc.VectorSubcoreMesh(core_axis_name="c", subcore_axis_name="s",
                                  num_cores=_NC, num_subcores=_NS)
    return pl.kernel(
        _sc_body,
        out_type=[
            jax.ShapeDtypeStruct((_R,), jnp.float32),
            jax.ShapeDtypeStruct((_R,), jnp.int32),
            jax.ShapeDtypeStruct((_R,), jnp.int32),
        ],
        mesh=mesh,
        scratch_types=[
            pltpu.VMEM((_C,), jnp.float32),
            pltpu.VMEM((_C,), jnp.float32),
            pltpu.VMEM((_RPW,), jnp.float32),
            pltpu.VMEM((_RPW,), jnp.int32),
            pltpu.VMEM((_L,), jnp.float32),
            pltpu.VMEM((_L,), jnp.int32),
            pltpu.VMEM((_L,), jnp.int32),
            pltpu.SemaphoreType.DMA,
            pltpu.SemaphoreType.DMA,
        ],
        interpret=interpret,
    )


@jax.jit
def kernel(class_log_probabilities, last_predictions, last_log_probabilities):
    B, beam = last_log_probabilities.shape
    lp_flat = last_log_probabilities.reshape(-1)
    outv, outc, outb = _make_sc_call()(
        class_log_probabilities, last_predictions, lp_flat)
    return (outv.reshape(B, beam), outc.reshape(B, beam),
            outb.reshape(B, beam))


# speedup vs baseline: 1.8229x; 1.1261x over previous
import jax
import jax.numpy as jnp
from jax import lax
from jax.experimental import pallas as pl
from jax.experimental.pallas import tpu as pltpu
from jax.experimental.pallas import tpu_sc as plsc

_NC, _NS, _L = 2, 16, 16
_NW = _NC * _NS
_R, _C = 512, 32768
_RPW = _R // _NW
_CHUNKS = _C // _L
_BEAM = 4
_EOS = 0
_NEG = float("-inf")
_LN2 = 0.6931471805599453


def _gather16(arr, idx):
    return arr.at[idx].get(mode="promise_in_bounds")


def _allmax(x):
    i = lax.iota(jnp.int32, 16)
    for d in (1, 2, 4, 8):
        x = jnp.maximum(x, _gather16(x, i ^ d))
    return x


def _allmin(x):
    i = lax.iota(jnp.int32, 16)
    for d in (1, 2, 4, 8):
        x = jnp.minimum(x, _gather16(x, i ^ d))
    return x


def _allsum(x):
    i = lax.iota(jnp.int32, 16)
    for d in (1, 2, 4, 8):
        x = x + _gather16(x, i ^ d)
    return x


def _ln_vec(s):
    bits = lax.bitcast_convert_type(s, jnp.int32)
    e = ((bits >> 23) & 0xFF) - 127
    mant = lax.bitcast_convert_type((bits & 0x7FFFFF) | 0x3F800000,
                                    jnp.float32)
    u = mant - 1.0
    y = e.astype(jnp.float32) * _LN2 + u * (1.0 + u * (-0.5 + u * (1.0 / 3.0)))
    for _ in range(3):
        y = y + s * jnp.exp(-y) - 1.0
    return y


def _process_row(buf, rlocal, acc):
    lse_a, rtv, rti = acc
    iota16 = lax.iota(jnp.int32, 16)

    def pb(j, c):
        s, t1, t2, t3, t4, i1, i2, i3, i4 = c
        x = buf[pl.ds(j * _L, _L)]
        s = s + jnp.exp(x)
        idx = j * _L + iota16
        b1 = x > t1
        b2 = x > t2
        b3 = x > t3
        b4 = x > t4
        t4n = jnp.where(b3, t3, jnp.where(b4, x, t4))
        i4n = jnp.where(b3, i3, jnp.where(b4, idx, i4))
        t3n = jnp.where(b2, t2, jnp.where(b3, x, t3))
        i3n = jnp.where(b2, i2, jnp.where(b3, idx, i3))
        t2n = jnp.where(b1, t1, jnp.where(b2, x, t2))
        i2n = jnp.where(b1, i1, jnp.where(b2, idx, i2))
        t1n = jnp.where(b1, x, t1)
        i1n = jnp.where(b1, idx, i1)
        return (s, t1n, t2n, t3n, t4n, i1n, i2n, i3n, i4n)

    zf = jnp.zeros((_L,), jnp.float32)
    nf = jnp.full((_L,), _NEG, jnp.float32)
    zi = jnp.zeros((_L,), jnp.int32)
    s, t1, t2, t3, t4, i1, i2, i3, i4 = lax.fori_loop(
        0, _CHUNKS, pb, (zf, nf, nf, nf, nf, zi, zi, zi, zi), unroll=4)

    S = _allsum(s)
    lse = _ln_vec(S)
    here = iota16 == rlocal
    lse_a = jnp.where(here, lse, lse_a)

    rtv_n = list(rtv)
    rti_n = list(rti)
    for k in range(4):
        mk = _allmax(t1)
        isel = _allmin(jnp.where(t1 == mk, i1, _C))
        rtv_n[k] = jnp.where(here, mk, rtv_n[k])
        rti_n[k] = jnp.where(here, isel, rti_n[k])
        w = (t1 == mk) & (i1 == isel)
        t1 = jnp.where(w, t2, t1)
        i1 = jnp.where(w, i2, i1)
        t2 = jnp.where(w, t3, t2)
        i2 = jnp.where(w, i3, i2)
        t3 = jnp.where(w, t4, t3)
        i3 = jnp.where(w, i4, i3)
        t4 = jnp.where(w, nf, t4)
    return (lse_a, tuple(rtv_n), tuple(rti_n))


def _sc_body(x_hbm, pred_hbm, lp_hbm, outv_hbm, outc_hbm, outb_hbm,
             buf0, buf1, lp_v, pred_v, ov, oc, ob, sem0, sem1):
    wid = lax.axis_index("s") * _NC + lax.axis_index("c")
    row0 = wid * _RPW
    pltpu.sync_copy(lp_hbm.at[pl.ds(row0, _RPW)], lp_v)
    pltpu.sync_copy(pred_hbm.at[pl.ds(row0, _RPW)], pred_v)

    iota16 = lax.iota(jnp.int32, 16)
    bufs = (buf0, buf1)
    sems = (sem0, sem1)
    pltpu.async_copy(x_hbm.at[row0], buf0, sem0)

    zf = jnp.zeros((_L,), jnp.float32)
    zi = jnp.zeros((_L,), jnp.int32)

    def group(g, acc):
        for b in range(2):
            r = g * 2 + b
            pltpu.make_async_copy(x_hbm.at[row0 + r], bufs[b], sems[b]).wait()

            @pl.when(r + 1 < _RPW)
            def _():
                pltpu.async_copy(x_hbm.at[row0 + r + 1], bufs[1 - b],
                                 sems[1 - b])
            lse_a, rtv, rti = _process_row(
                bufs[b], r, (acc[0], acc[1:5], acc[5:9]))
            acc = (lse_a,) + rtv + rti
        return acc

    acc = lax.fori_loop(0, _RPW // 2, group,
                        (zf, zf, zf, zf, zf, zi, zi, zi, zi))
    lse_a = acc[0]
    rtv = acc[1:5]
    rti = acc[5:9]
    lp = lp_v[...]
    pred = pred_v[...]

    outv = jnp.zeros((_L,), jnp.float32)
    outc = jnp.zeros((_L,), jnp.int32)
    outb = jnp.zeros((_L,), jnp.int32)
    k_in = iota16 & 3
    for b in range(4):
        rsel = b * 4 + (iota16 >> 2)
        gv = [_gather16(rtv[k], rsel) for k in range(4)]
        gi = [_gather16(rti[k], rsel) for k in range(4)]
        v = jnp.where(k_in == 0, gv[0],
                      jnp.where(k_in == 1, gv[1],
                                jnp.where(k_in == 2, gv[2], gv[3])))
        ci = jnp.where(k_in == 0, gi[0],
                       jnp.where(k_in == 1, gi[1],
                                 jnp.where(k_in == 2, gi[2], gi[3])))
        lpg = _gather16(lp, rsel)
        lseg = _gather16(lse_a, rsel)
        predg = _gather16(pred, rsel)
        fin = predg == _EOS
        candv = jnp.where(fin,
                          jnp.where(k_in == 0, lpg,
                                    jnp.full((16,), _NEG, jnp.float32)),
                          lpg + v - lseg)
        candc = jnp.where(fin, 0, ci)
        flat = ((iota16 >> 2) << 15) | candc
        for k in range(4):
            mk = _allmax(candv)
            fsel = _allmin(jnp.where(candv == mk, flat, _BEAM * _C))
            candv = jnp.where(flat == fsel, _NEG, candv)
            here = iota16 == (b * 4 + k)
            outv = jnp.where(here, mk, outv)
            outc = jnp.where(here, fsel & (_C - 1), outc)
            outb = jnp.where(here, fsel >> 15, outb)

    ov[...] = outv
    oc[...] = outc
    ob[...] = outb
    pltpu.sync_copy(ov, outv_hbm.at[pl.ds(row0, _RPW)])
    pltpu.sync_copy(oc, outc_hbm.at[pl.ds(row0, _RPW)])
    pltpu.sync_copy(ob, outb_hbm.at[pl.ds(row0, _RPW)])


def _make_sc_call(interpret=False):
    mesh = plsc.VectorSubcoreMesh(core_axis_name="c", subcore_axis_name="s",
                                  num_cores=_NC, num_subcores=_NS)
    return pl.kernel(
        _sc_body,
        out_type=[
            jax.ShapeDtypeStruct((_R,), jnp.float32),
            jax.ShapeDtypeStruct((_R,), jnp.int32),
            jax.ShapeDtypeStruct((_R,), jnp.int32),
        ],
        mesh=mesh,
        scratch_types=[
            pltpu.VMEM((_C,), jnp.float32),
            pltpu.VMEM((_C,), jnp.float32),
            pltpu.VMEM((_RPW,), jnp.float32),
            pltpu.VMEM((_RPW,), jnp.int32),
            pltpu.VMEM((_L,), jnp.float32),
            pltpu.VMEM((_L,), jnp.int32),
            pltpu.VMEM((_L,), jnp.int32),
            pltpu.SemaphoreType.DMA,
            pltpu.SemaphoreType.DMA,
        ],
        interpret=interpret,
    )


@jax.jit
def kernel(class_log_probabilities, last_predictions, last_log_probabilities):
    B, beam = last_log_probabilities.shape
    lp_flat = last_log_probabilities.reshape(-1)
    outv, outc, outb = _make_sc_call()(
        class_log_probabilities, last_predictions, lp_flat)
    return (outv.reshape(B, beam), outc.reshape(B, beam),
            outb.reshape(B, beam))


# speedup vs baseline: 3.2676x; 1.7925x over previous
import functools

import jax
import jax.numpy as jnp
from jax import lax
from jax.experimental import pallas as pl
from jax.experimental.pallas import tpu as pltpu
from jax.experimental.pallas import tpu_sc as plsc

_NC, _NS, _L = 2, 16, 16
_NW = _NC * _NS
_R, _C = 512, 32768
_B = 128
_BEAM = 4
_EOS = 0
_NEG = float("-inf")
_LN2 = 0.6931471805599453

_SC_BATCHES = _NW
_SC_ROWS = _SC_BATCHES * _BEAM
_TC_ROWS = _R - _SC_ROWS
_TC_B = _TC_ROWS // _BEAM
_ROW0_SC = _TC_ROWS
_CHUNKS = _C // _L
_RPW = _BEAM


def _gather16(arr, idx):
    return arr.at[idx].get(mode="promise_in_bounds")


def _allmax(x):
    i = lax.iota(jnp.int32, 16)
    for d in (1, 2, 4, 8):
        x = jnp.maximum(x, _gather16(x, i ^ d))
    return x


def _allmin(x):
    i = lax.iota(jnp.int32, 16)
    for d in (1, 2, 4, 8):
        x = jnp.minimum(x, _gather16(x, i ^ d))
    return x


def _allsum(x):
    i = lax.iota(jnp.int32, 16)
    for d in (1, 2, 4, 8):
        x = x + _gather16(x, i ^ d)
    return x


def _ln_vec(s):
    bits = lax.bitcast_convert_type(s, jnp.int32)
    e = ((bits >> 23) & 0xFF) - 127
    mant = lax.bitcast_convert_type((bits & 0x7FFFFF) | 0x3F800000,
                                    jnp.float32)
    u = mant - 1.0
    y = e.astype(jnp.float32) * _LN2 + u * (1.0 + u * (-0.5 + u * (1.0 / 3.0)))
    for _ in range(3):
        y = y + s * jnp.exp(-y) - 1.0
    return y


def _sc_process_row(buf, rlocal, acc):
    lse_a, rtv, rti = acc
    iota16 = lax.iota(jnp.int32, 16)

    def pb(j, c):
        s, t1, t2, t3, t4, i1, i2, i3, i4 = c
        x = buf[pl.ds(j * _L, _L)]
        s = s + jnp.exp(x)
        idx = j * _L + iota16
        b1 = x > t1
        b2 = x > t2
        b3 = x > t3
        b4 = x > t4
        t4n = jnp.where(b3, t3, jnp.where(b4, x, t4))
        i4n = jnp.where(b3, i3, jnp.where(b4, idx, i4))
        t3n = jnp.where(b2, t2, jnp.where(b3, x, t3))
        i3n = jnp.where(b2, i2, jnp.where(b3, idx, i3))
        t2n = jnp.where(b1, t1, jnp.where(b2, x, t2))
        i2n = jnp.where(b1, i1, jnp.where(b2, idx, i2))
        t1n = jnp.where(b1, x, t1)
        i1n = jnp.where(b1, idx, i1)
        return (s, t1n, t2n, t3n, t4n, i1n, i2n, i3n, i4n)

    zf = jnp.zeros((_L,), jnp.float32)
    nf = jnp.full((_L,), _NEG, jnp.float32)
    zi = jnp.zeros((_L,), jnp.int32)
    s, t1, t2, t3, t4, i1, i2, i3, i4 = lax.fori_loop(
        0, _CHUNKS, pb, (zf, nf, nf, nf, nf, zi, zi, zi, zi), unroll=4)

    S = _allsum(s)
    lse = _ln_vec(S)
    here = iota16 == rlocal
    lse_a = jnp.where(here, lse, lse_a)

    rtv_n = list(rtv)
    rti_n = list(rti)
    for k in range(4):
        mk = _allmax(t1)
        isel = _allmin(jnp.where(t1 == mk, i1, _C))
        rtv_n[k] = jnp.where(here, mk, rtv_n[k])
        rti_n[k] = jnp.where(here, isel, rti_n[k])
        w = (t1 == mk) & (i1 == isel)
        t1 = jnp.where(w, t2, t1)
        i1 = jnp.where(w, i2, i1)
        t2 = jnp.where(w, t3, t2)
        i2 = jnp.where(w, i3, i2)
        t3 = jnp.where(w, t4, t3)
        i3 = jnp.where(w, i4, i3)
        t4 = jnp.where(w, nf, t4)
    return (lse_a, tuple(rtv_n), tuple(rti_n))


def _sc_body(x_hbm, lpp_hbm, out_hbm, buf0, buf1, lpp_v, ov, sem0, sem1):
    wid = lax.axis_index("s") * _NC + lax.axis_index("c")
    batch = _TC_B + wid
    row0 = batch * _BEAM
    pltpu.sync_copy(lpp_hbm.at[batch], lpp_v)

    iota16 = lax.iota(jnp.int32, 16)
    bufs = (buf0, buf1)
    sems = (sem0, sem1)
    pltpu.async_copy(x_hbm.at[row0], buf0, sem0)

    zf = jnp.zeros((_L,), jnp.float32)
    zi = jnp.zeros((_L,), jnp.int32)

    def group(g, acc):
        for b in range(2):
            r = g * 2 + b
            pltpu.make_async_copy(x_hbm.at[row0 + r], bufs[b], sems[b]).wait()

            @pl.when(r + 1 < _RPW)
            def _():
                pltpu.async_copy(x_hbm.at[row0 + r + 1], bufs[1 - b],
                                 sems[1 - b])
            lse_a, rtv, rti = _sc_process_row(
                bufs[b], r, (acc[0], acc[1:5], acc[5:9]))
            acc = (lse_a,) + rtv + rti
        return acc

    acc = lax.fori_loop(0, _RPW // 2, group,
                        (zf, zf, zf, zf, zf, zi, zi, zi, zi))
    lse_a = acc[0]
    rtv = acc[1:5]
    rti = acc[5:9]
    lpp = lpp_v[...]

    k_in = iota16 & 3
    rsel = iota16 >> 2
    gv = [_gather16(rtv[k], rsel) for k in range(4)]
    gi = [_gather16(rti[k], rsel) for k in range(4)]
    v = jnp.where(k_in == 0, gv[0],
                  jnp.where(k_in == 1, gv[1],
                            jnp.where(k_in == 2, gv[2], gv[3])))
    ci = jnp.where(k_in == 0, gi[0],
                   jnp.where(k_in == 1, gi[1],
                             jnp.where(k_in == 2, gi[2], gi[3])))
    lpg = _gather16(lpp, rsel)
    lseg = _gather16(lse_a, rsel)
    predg = _gather16(lpp, 4 + rsel)
    fin = predg == jnp.float32(_EOS)
    candv = jnp.where(fin,
                      jnp.where(k_in == 0, lpg,
                                jnp.full((16,), _NEG, jnp.float32)),
                      lpg + (v - lseg))
    candc = jnp.where(fin, 0, ci)
    flat = ((iota16 >> 2) << 15) | candc

    pack = jnp.zeros((_L,), jnp.float32)
    for k in range(4):
        mk = _allmax(candv)
        fsel = _allmin(jnp.where(candv == mk, flat, _BEAM * _C))
        candv = jnp.where(flat == fsel, _NEG, candv)
        clsf = (fsel & (_C - 1)).astype(jnp.float32)
        bpf = (fsel >> 15).astype(jnp.float32)
        pack = jnp.where(iota16 == k, mk, pack)
        pack = jnp.where(iota16 == 4 + k, clsf, pack)
        pack = jnp.where(iota16 == 8 + k, bpf, pack)

    ov[...] = pack
    pltpu.sync_copy(ov, out_hbm.at[wid])


def _make_sc_call():
    mesh = plsc.VectorSubcoreMesh(core_axis_name="c", subcore_axis_name="s",
                                  num_cores=_NC, num_subcores=_NS)
    return pl.kernel(
        _sc_body,
        out_type=jax.ShapeDtypeStruct((_SC_BATCHES, _L), jnp.float32),
        mesh=mesh,
        scratch_types=[
            pltpu.VMEM((_C,), jnp.float32),
            pltpu.VMEM((_C,), jnp.float32),
            pltpu.VMEM((_L,), jnp.float32),
            pltpu.VMEM((_L,), jnp.float32),
            pltpu.SemaphoreType.DMA,
            pltpu.SemaphoreType.DMA,
        ],
    )


def _scan_body(x_ref, topv_ref, topi_ref, lse_ref):
    x = x_ref[...]
    rows, C = x.shape
    m = jnp.max(x, axis=1, keepdims=True)
    s = jnp.sum(jnp.exp(x - m), axis=1, keepdims=True)
    lse_ref[...] = m + jnp.log(s)

    iota = lax.broadcasted_iota(jnp.int32, x.shape, 1)
    xc = x
    vals = []
    idxs = []
    for _ in range(_BEAM):
        mk = jnp.max(xc, axis=1, keepdims=True)
        ik = jnp.min(jnp.where(xc == mk, iota, C), axis=1, keepdims=True)
        vals.append(mk)
        idxs.append(ik)
        xc = jnp.where(iota == ik, _NEG, xc)
    topv_ref[...] = jnp.concatenate(vals, axis=1)
    topi_ref[...] = jnp.concatenate(idxs, axis=1)


def _merge_body(C, topv_ref, topi_ref, lse16_ref, lp16_ref, pred16_ref,
                outv_ref, outc_ref, outb_ref):
    topv = topv_ref[...]
    topi = topi_ref[...]
    lse16 = lse16_ref[...]
    lp16 = lp16_ref[...]
    pred16 = pred16_ref[...]
    B = topv.shape[0]

    lane = lax.broadcasted_iota(jnp.int32, (B, 16), 1)
    beam = lane // _BEAM
    k_in_beam = lane % _BEAM
    finished = pred16 == _EOS

    base_v = lp16 + (topv - lse16)
    fin_v = jnp.where(k_in_beam == 0, lp16, _NEG)
    cand_v = jnp.where(finished, fin_v, base_v)
    cand_c = jnp.where(finished, 0, topi)
    flat = beam * C + cand_c

    big = _BEAM * C
    outv = []
    outc = []
    outb = []
    for _ in range(_BEAM):
        mk = jnp.max(cand_v, axis=1, keepdims=True)
        fsel = jnp.min(jnp.where(cand_v == mk, flat, big), axis=1,
                       keepdims=True)
        outv.append(mk)
        outb.append(fsel // C)
        outc.append(fsel % C)
        cand_v = jnp.where(flat == fsel, _NEG, cand_v)
    outv_ref[...] = jnp.concatenate(outv, axis=1)
    outc_ref[...] = jnp.concatenate(outc, axis=1)
    outb_ref[...] = jnp.concatenate(outb, axis=1)


@jax.jit
def kernel(class_log_probabilities, last_predictions, last_log_probabilities):
    B, beam = last_log_probabilities.shape
    C = _C
    ROWS_PER_STEP = 64
    steps = _TC_ROWS // ROWS_PER_STEP

    predf = last_predictions.reshape(B, beam).astype(jnp.float32)
    lpp = jnp.concatenate(
        [last_log_probabilities, predf,
         jnp.zeros((B, 8), jnp.float32)], axis=1)

    sc_out = _make_sc_call()(class_log_probabilities, lpp)

    topv, topi, lse = pl.pallas_call(
        _scan_body,
        grid=(steps,),
        in_specs=[pl.BlockSpec((ROWS_PER_STEP, C), lambda i: (i, 0))],
        out_specs=[
            pl.BlockSpec((ROWS_PER_STEP, beam), lambda i: (i, 0)),
            pl.BlockSpec((ROWS_PER_STEP, beam), lambda i: (i, 0)),
            pl.BlockSpec((ROWS_PER_STEP, 1), lambda i: (i, 0)),
        ],
        out_shape=[
            jax.ShapeDtypeStruct((_TC_ROWS, beam), jnp.float32),
            jax.ShapeDtypeStruct((_TC_ROWS, beam), jnp.int32),
            jax.ShapeDtypeStruct((_TC_ROWS, 1), jnp.float32),
        ],
    )(class_log_probabilities)

    topv16 = topv.reshape(_TC_B, beam * _BEAM)
    topi16 = topi.reshape(_TC_B, beam * _BEAM)
    lse16 = jnp.repeat(lse.reshape(_TC_B, beam), _BEAM, axis=1)
    lp16 = jnp.repeat(last_log_probabilities[:_TC_B], _BEAM, axis=1)
    pred16 = jnp.repeat(
        last_predictions[:_TC_ROWS].reshape(_TC_B, beam), _BEAM, axis=1)

    tc_outv, tc_outc, tc_outb = pl.pallas_call(
        functools.partial(_merge_body, C),
        out_shape=[
            jax.ShapeDtypeStruct((_TC_B, beam), jnp.float32),
            jax.ShapeDtypeStruct((_TC_B, beam), jnp.int32),
            jax.ShapeDtypeStruct((_TC_B, beam), jnp.int32),
        ],
    )(topv16, topi16, lse16, lp16, pred16)

    sc_v = sc_out[:, 0:4]
    sc_c = sc_out[:, 4:8].astype(jnp.int32)
    sc_b = sc_out[:, 8:12].astype(jnp.int32)

    outv = jnp.concatenate([tc_outv, sc_v], axis=0)
    outc = jnp.concatenate([tc_outc, sc_c], axis=0)
    outb = jnp.concatenate([tc_outb, sc_b], axis=0)
    return outv, outc, outb
